# baseline (speedup 1.0000x reference)
import functools

import jax
import jax.numpy as jnp
from jax import lax
from jax.experimental import pallas as pl
from jax.experimental.pallas import tpu as pltpu
from jax.experimental.pallas import tpu_sc as plsc


def _asf32(x):
    return jax.lax.bitcast_convert_type(x, jnp.float32)

B, L, INTD, HD = 4096, 200, 64, 192
N = B * L
HDP = 256

NC, NS = 2, 16
NW = NC * NS
CHUNK = 64
G = 8
LANES = 16

N_TEST, N_Q, N_TAG, N_EL, N_INT = 1001, 100001, 1001, 301, 3
ST_A = 1008
ST_B = 304


WORDS = HDP // 2


def _pack_words(x):
    xb = x.astype(jnp.bfloat16)
    lo = jax.lax.bitcast_convert_type(xb[:, :WORDS], jnp.uint16).astype(jnp.int32)
    hi = jax.lax.bitcast_convert_type(xb[:, WORDS:], jnp.uint16).astype(jnp.int32)
    return lo | (hi << 16)


def _proj(E, Wk, bias, out_dtype=jnp.float32):
    n, d = E.shape
    bm = min(n, 512)
    grid = (pl.cdiv(n, bm),)
    pack = out_dtype == jnp.int32
    ncols = WORDS if pack else HDP

    def body(e_ref, w_ref, b_ref, o_ref):
        acc = (
            jnp.dot(e_ref[...], w_ref[...], preferred_element_type=jnp.float32)
            + b_ref[...]
        )
        o_ref[...] = _pack_words(acc) if pack else acc

    return pl.pallas_call(
        body,
        grid=grid,
        in_specs=[
            pl.BlockSpec((bm, d), lambda i: (i, 0)),
            pl.BlockSpec((d, HDP), lambda i: (0, 0)),
            pl.BlockSpec((1, HDP), lambda i: (0, 0)),
        ],
        out_specs=pl.BlockSpec((bm, ncols), lambda i: (i, 0)),
        out_shape=jax.ShapeDtypeStruct((n, ncols), out_dtype),
    )(E, Wk, bias)


def _pair_table(P_big, P_one, stride, n_one):
    def body(pb_ref, po_ref, o_ref):
        o_ref[...] = _pack_words(pb_ref[...] + po_ref[0])

    return pl.pallas_call(
        body,
        grid=(n_one,),
        in_specs=[
            pl.BlockSpec((stride, HDP), lambda i: (0, 0)),
            pl.BlockSpec((1, 1, HDP), lambda i: (i, 0, 0)),
        ],
        out_specs=pl.BlockSpec((stride, WORDS), lambda i: (i, 0)),
        out_shape=jax.ShapeDtypeStruct((n_one * stride, WORDS), jnp.int32),
    )(P_big, P_one.reshape(n_one, 1, HDP))


def _make_gather_sum():
    per_w = N // NW
    sup_len = G * CHUNK
    sups = per_w // sup_len
    mesh = plsc.VectorSubcoreMesh(core_axis_name="c", subcore_axis_name="s")

    @functools.partial(
        pl.kernel,
        out_type=jax.ShapeDtypeStruct((N, HD), jnp.float32),
        mesh=mesh,
        scratch_types=[
            pltpu.VMEM((G, CHUNK), jnp.int32),
            pltpu.VMEM((G, CHUNK), jnp.int32),
            pltpu.VMEM((G, CHUNK), jnp.int32),
            pltpu.VMEM((G, CHUNK), jnp.int32),
            pltpu.VMEM((G, CHUNK), jnp.int32),
            pltpu.VMEM((2, CHUNK, WORDS), jnp.int32),
            pltpu.VMEM((2, CHUNK, WORDS), jnp.int32),
            pltpu.VMEM((2, CHUNK, WORDS), jnp.int32),
            pltpu.VMEM((CHUNK, HD), jnp.float32),
            pltpu.SemaphoreType.DMA,
            pltpu.SemaphoreType.DMA,
            pltpu.SemaphoreType.DMA,
        ],
    )
    def gather_sum(tA, tB, tQ, iInt, iTest, iTag, iEl, iQ, out,
                   rInt, rTest, rTag, rEl, rQ,
                   gA, gB, gQ, ov, sg0, sg1, so):
        wid = lax.axis_index("s") * NC + lax.axis_index("c")
        base0 = wid * per_w
        sgs = (sg0, sg1)

        def issue(j, slot):
            pltpu.async_copy(tA.at[rInt.at[j]], gA.at[slot], sgs[slot])
            pltpu.async_copy(tB.at[rTag.at[j]], gB.at[slot], sgs[slot])
            pltpu.async_copy(tQ.at[rQ.at[j]], gQ.at[slot], sgs[slot])

        def wait_gather(j, slot):
            pltpu.make_async_copy(tA.at[rInt.at[j]], gA.at[slot], sgs[slot]).wait()
            pltpu.make_async_copy(tB.at[rTag.at[j]], gB.at[slot], sgs[slot]).wait()
            pltpu.make_async_copy(tQ.at[rQ.at[j]], gQ.at[slot], sgs[slot]).wait()

        def out_slice(chunk_id):
            return out.at[pl.ds(base0 + chunk_id * CHUNK, CHUNK)]

        def sup_body(s, carry):
            srow = pl.multiple_of((base0 // CHUNK) + s * G, 8)
            sl_idx = pl.ds(srow, G)
            pltpu.sync_copy(iInt.at[sl_idx], rInt)
            pltpu.sync_copy(iTest.at[sl_idx], rTest)
            pltpu.sync_copy(iTag.at[sl_idx], rTag)
            pltpu.sync_copy(iEl.at[sl_idx], rEl)
            pltpu.sync_copy(iQ.at[sl_idx], rQ)

            def comb_body(g, carry2):
                for v in range(CHUNK // LANES):
                    sl = pl.ds(v * LANES, LANES)
                    rInt[g, sl] = rInt[g, sl] * ST_A + rTest[g, sl]
                    rTag[g, sl] = rTag[g, sl] * ST_B + rEl[g, sl]
                return carry2

            lax.fori_loop(0, G, comb_body, 0, unroll=False)

            issue(0, 0)

            def pair_body(m, carry2):
                for bslot in range(2):
                    j = 2 * m + bslot
                    jg = s * G + j
                    if bslot == 0:
                        issue(j + 1, 1)
                    else:

                        @pl.when(j + 1 < G)
                        def _():
                            issue(j + 1, 0)

                    wait_gather(j, bslot)

                    @pl.when(jg >= 1)
                    def _():
                        prev = jnp.maximum(jg - 1, 0)
                        pltpu.make_async_copy(ov, out_slice(prev), so).wait()

                    def sum_body(c, carry3):
                        mask = jnp.int32(-65536)
                        for v in range(WORDS // LANES):
                            sl = pl.ds(v * LANES, LANES)
                            a = gA[bslot, c, sl]
                            bw = gB[bslot, c, sl]
                            q = gQ[bslot, c, sl]
                            lo = (_asf32(a << 16) + _asf32(bw << 16)
                                  + _asf32(q << 16))
                            ov[c, pl.ds(v * LANES, LANES)] = lo
                            if v < (HD - WORDS) // LANES:
                                hi = (_asf32(a & mask) + _asf32(bw & mask)
                                      + _asf32(q & mask))
                                ov[c, pl.ds(WORDS + v * LANES, LANES)] = hi
                        return carry3

                    lax.fori_loop(0, CHUNK, sum_body, 0, unroll=False)
                    pltpu.async_copy(ov, out_slice(jg), so)
                return carry2

            lax.fori_loop(0, G // 2, pair_body, 0, unroll=False)
            return carry

        lax.fori_loop(0, sups, sup_body, 0, unroll=False)

        pltpu.make_async_copy(ov, out_slice(sups * G - 1), so).wait()

    return gather_sum


_gather_sum = _make_gather_sum()


def kernel(test, question, tag, correct, elapsed_question, mask, interaction,
           extra, E_int, E_test, E_q, E_tag, E_el, W, b):
    pad = ((0, 0), (0, HDP - HD))
    zero = jnp.zeros((1, HDP), jnp.float32)
    bias = jnp.pad(b.reshape(1, HD), pad)
    Wp = [jnp.pad(W[k * INTD:(k + 1) * INTD], pad) for k in range(5)]

    P_int = _proj(E_int, Wp[0], bias)
    P_test = _proj(E_test, Wp[1], zero)
    P_tag = _proj(E_tag, Wp[3], zero)
    P_el = _proj(E_el, Wp[4], zero)
    P_test_p = jnp.pad(P_test, ((0, ST_A - N_TEST), (0, 0)))
    P_el_p = jnp.pad(P_el, ((0, ST_B - N_EL), (0, 0)))
    tab_A = _pair_table(P_test_p, P_int, ST_A, N_INT)
    tab_B = _pair_table(P_el_p, P_tag, ST_B, N_TAG)
    tab_Q = _proj(E_q, Wp[2], zero, out_dtype=jnp.int32)

    i_int = interaction.reshape(N // CHUNK, CHUNK).astype(jnp.int32)
    i_test = test.reshape(N // CHUNK, CHUNK).astype(jnp.int32)
    i_q = question.reshape(N // CHUNK, CHUNK).astype(jnp.int32)
    i_tag = tag.reshape(N // CHUNK, CHUNK).astype(jnp.int32)
    i_el = elapsed_question.reshape(N // CHUNK, CHUNK).astype(jnp.int32)

    out = _gather_sum(tab_A, tab_B, tab_Q,
                      i_int, i_test, i_tag, i_el, i_q)
    return out.reshape(B, L, HD)


# speedup vs baseline: 1.4590x; 1.4590x over previous
import functools

import jax
import jax.numpy as jnp
from jax import lax
from jax.experimental import pallas as pl
from jax.experimental.pallas import tpu as pltpu
from jax.experimental.pallas import tpu_sc as plsc


def _asf32(x):
    return jax.lax.bitcast_convert_type(x, jnp.float32)

B, L, INTD, HD = 4096, 200, 64, 192
N = B * L
HDP = 256

NC, NS = 2, 16
NW = NC * NS
CHUNK = 64
G = 8
LANES = 16

N_TEST, N_Q, N_TAG, N_EL, N_INT = 1001, 100001, 1001, 301, 3
ST_A = 1008
ST_B = 304


WORDS = HDP // 2


def _pack_words(x):
    xb = x.astype(jnp.bfloat16)
    lo = jax.lax.bitcast_convert_type(xb[:, :WORDS], jnp.uint16).astype(jnp.int32)
    hi = jax.lax.bitcast_convert_type(xb[:, WORDS:], jnp.uint16).astype(jnp.int32)
    return lo | (hi << 16)


def _proj(E, Wk, bias, out_dtype=jnp.float32):
    n, d = E.shape
    bm = min(n, 512)
    grid = (pl.cdiv(n, bm),)
    pack = out_dtype == jnp.int32
    ncols = WORDS if pack else HDP

    def body(e_ref, w_ref, b_ref, o_ref):
        acc = (
            jnp.dot(e_ref[...], w_ref[...], preferred_element_type=jnp.float32)
            + b_ref[...]
        )
        o_ref[...] = _pack_words(acc) if pack else acc

    return pl.pallas_call(
        body,
        grid=grid,
        in_specs=[
            pl.BlockSpec((bm, d), lambda i: (i, 0)),
            pl.BlockSpec((d, HDP), lambda i: (0, 0)),
            pl.BlockSpec((1, HDP), lambda i: (0, 0)),
        ],
        out_specs=pl.BlockSpec((bm, ncols), lambda i: (i, 0)),
        out_shape=jax.ShapeDtypeStruct((n, ncols), out_dtype),
    )(E, Wk, bias)


def _pair_table(P_big, P_one, stride, n_one):
    def body(pb_ref, po_ref, o_ref):
        o_ref[...] = _pack_words(pb_ref[...] + po_ref[0])

    return pl.pallas_call(
        body,
        grid=(n_one,),
        in_specs=[
            pl.BlockSpec((stride, HDP), lambda i: (0, 0)),
            pl.BlockSpec((1, 1, HDP), lambda i: (i, 0, 0)),
        ],
        out_specs=pl.BlockSpec((stride, WORDS), lambda i: (i, 0)),
        out_shape=jax.ShapeDtypeStruct((n_one * stride, WORDS), jnp.int32),
    )(P_big, P_one.reshape(n_one, 1, HDP))


def _make_gather_sum():
    per_w = N // NW
    sup_len = G * CHUNK
    sups = per_w // sup_len
    mesh = plsc.VectorSubcoreMesh(core_axis_name="c", subcore_axis_name="s")

    @functools.partial(
        pl.kernel,
        out_type=jax.ShapeDtypeStruct((N, HD), jnp.float32),
        mesh=mesh,
        scratch_types=[
            pltpu.VMEM((G, CHUNK), jnp.int32),
            pltpu.VMEM((G, CHUNK), jnp.int32),
            pltpu.VMEM((G, CHUNK), jnp.int32),
            pltpu.VMEM((G, CHUNK), jnp.int32),
            pltpu.VMEM((G, CHUNK), jnp.int32),
            pltpu.VMEM((2, CHUNK, WORDS), jnp.int32),
            pltpu.VMEM((2, CHUNK, WORDS), jnp.int32),
            pltpu.VMEM((2, CHUNK, WORDS), jnp.int32),
            pltpu.VMEM((CHUNK, HD), jnp.float32),
            pltpu.SemaphoreType.DMA,
            pltpu.SemaphoreType.DMA,
            pltpu.SemaphoreType.DMA,
        ],
    )
    def gather_sum(tA, tB, tQ, iInt, iTest, iTag, iEl, iQ, out,
                   rInt, rTest, rTag, rEl, rQ,
                   gA, gB, gQ, ov, sg0, sg1, so):
        wid = lax.axis_index("s") * NC + lax.axis_index("c")
        base0 = wid * per_w
        sgs = (sg0, sg1)

        def issue(j, slot):
            pltpu.async_copy(tA.at[rInt.at[j]], gA.at[slot], sgs[slot])
            pltpu.async_copy(tB.at[rTag.at[j]], gB.at[slot], sgs[slot])
            pltpu.async_copy(tQ.at[rQ.at[j]], gQ.at[slot], sgs[slot])

        def wait_gather(j, slot):
            pltpu.make_async_copy(tA.at[rInt.at[j]], gA.at[slot], sgs[slot]).wait()
            pltpu.make_async_copy(tB.at[rTag.at[j]], gB.at[slot], sgs[slot]).wait()
            pltpu.make_async_copy(tQ.at[rQ.at[j]], gQ.at[slot], sgs[slot]).wait()

        def out_slice(chunk_id):
            return out.at[pl.ds(base0 + chunk_id * CHUNK, CHUNK)]

        def sup_body(s, carry):
            srow = pl.multiple_of((base0 // CHUNK) + s * G, 8)
            sl_idx = pl.ds(srow, G)
            pltpu.sync_copy(iInt.at[sl_idx], rInt)
            pltpu.sync_copy(iTest.at[sl_idx], rTest)
            pltpu.sync_copy(iTag.at[sl_idx], rTag)
            pltpu.sync_copy(iEl.at[sl_idx], rEl)
            pltpu.sync_copy(iQ.at[sl_idx], rQ)

            def comb_body(g, carry2):
                for v in range(CHUNK // LANES):
                    sl = pl.ds(v * LANES, LANES)
                    rInt[g, sl] = rInt[g, sl] * ST_A + rTest[g, sl]
                    rTag[g, sl] = rTag[g, sl] * ST_B + rEl[g, sl]
                return carry2

            lax.fori_loop(0, G, comb_body, 0, unroll=False)

            issue(0, 0)

            def pair_body(m, carry2):
                for bslot in range(2):
                    j = 2 * m + bslot
                    jg = s * G + j
                    if bslot == 0:
                        issue(j + 1, 1)
                    else:

                        @pl.when(j + 1 < G)
                        def _():
                            issue(j + 1, 0)

                    wait_gather(j, bslot)

                    @pl.when(jg >= 1)
                    def _():
                        prev = jnp.maximum(jg - 1, 0)
                        pltpu.make_async_copy(ov, out_slice(prev), so).wait()

                    @plsc.parallel_loop(0, CHUNK, unroll=4)
                    def _(c):
                        mask = jnp.int32(-65536)
                        for v in range(WORDS // LANES):
                            sl = pl.ds(v * LANES, LANES)
                            a = gA[bslot, c, sl]
                            bw = gB[bslot, c, sl]
                            q = gQ[bslot, c, sl]
                            lo = (_asf32(a << 16) + _asf32(bw << 16)
                                  + _asf32(q << 16))
                            ov[c, pl.ds(v * LANES, LANES)] = lo
                            if v < (HD - WORDS) // LANES:
                                hi = (_asf32(a & mask) + _asf32(bw & mask)
                                      + _asf32(q & mask))
                                ov[c, pl.ds(WORDS + v * LANES, LANES)] = hi
                    pltpu.async_copy(ov, out_slice(jg), so)
                return carry2

            lax.fori_loop(0, G // 2, pair_body, 0, unroll=False)
            return carry

        lax.fori_loop(0, sups, sup_body, 0, unroll=False)

        pltpu.make_async_copy(ov, out_slice(sups * G - 1), so).wait()

    return gather_sum


_gather_sum = _make_gather_sum()


def kernel(test, question, tag, correct, elapsed_question, mask, interaction,
           extra, E_int, E_test, E_q, E_tag, E_el, W, b):
    pad = ((0, 0), (0, HDP - HD))
    zero = jnp.zeros((1, HDP), jnp.float32)
    bias = jnp.pad(b.reshape(1, HD), pad)
    Wp = [jnp.pad(W[k * INTD:(k + 1) * INTD], pad) for k in range(5)]

    P_int = _proj(E_int, Wp[0], bias)
    P_test = _proj(E_test, Wp[1], zero)
    P_tag = _proj(E_tag, Wp[3], zero)
    P_el = _proj(E_el, Wp[4], zero)
    P_test_p = jnp.pad(P_test, ((0, ST_A - N_TEST), (0, 0)))
    P_el_p = jnp.pad(P_el, ((0, ST_B - N_EL), (0, 0)))
    tab_A = _pair_table(P_test_p, P_int, ST_A, N_INT)
    tab_B = _pair_table(P_el_p, P_tag, ST_B, N_TAG)
    tab_Q = _proj(E_q, Wp[2], zero, out_dtype=jnp.int32)

    i_int = interaction.reshape(N // CHUNK, CHUNK).astype(jnp.int32)
    i_test = test.reshape(N // CHUNK, CHUNK).astype(jnp.int32)
    i_q = question.reshape(N // CHUNK, CHUNK).astype(jnp.int32)
    i_tag = tag.reshape(N // CHUNK, CHUNK).astype(jnp.int32)
    i_el = elapsed_question.reshape(N // CHUNK, CHUNK).astype(jnp.int32)

    out = _gather_sum(tab_A, tab_B, tab_Q,
                      i_int, i_test, i_tag, i_el, i_q)
    return out.reshape(B, L, HD)
